# Initial kernel scaffold; baseline (speedup 1.0000x reference)
import jax
import jax.numpy as jnp
from jax.experimental import pallas as pl


def kernel(h, segment_ids, attention_query, w):
    raise NotImplementedError("write your pallas kernel here")


# speedup vs baseline: 13.5316x; 13.5316x over previous
import jax
import jax.numpy as jnp
from jax.experimental import pallas as pl
from jax.experimental.pallas import tpu as pltpu

_HID = 128
_NSEG = 64
_CHUNK = 20000
_NEG = -3.0e38


def _body(seg_ref, h_ref, wr_ref, q_ref, out_ref, m_ref, s_ref, acc_ref):
    i = pl.program_id(0)
    k = pl.num_programs(0)

    @pl.when(i == 0)
    def _():
        m_ref[...] = jnp.full((1, _NSEG), _NEG, jnp.float32)
        s_ref[...] = jnp.zeros((1, _NSEG), jnp.float32)
        acc_ref[...] = jnp.zeros((_NSEG, _HID), jnp.float32)

    h = h_ref[...]
    seg = seg_ref[0, 0, :]
    wr = wr_ref[...]
    q = q_ref[...]

    qd = jax.lax.dot_general(q, wr, (((1,), (0,)), ((), ())),
                             preferred_element_type=jnp.float32)

    ids = jax.lax.broadcasted_iota(jnp.int32, (_CHUNK, _NSEG), 1)
    oh = seg[:, None] == ids
    ohf = oh.astype(jnp.float32)

    lhs = jnp.concatenate([h, ohf], axis=1)
    rhs = jnp.concatenate([wr[:, 0:1], qd[:, 1:2]], axis=0)
    g = jax.lax.dot_general(lhs, rhs, (((1,), (0,)), ((), ())),
                            preferred_element_type=jnp.float32)[:, 0]
    g = jnp.where(g >= 0, g, 0.01 * g)

    masked = jnp.where(oh, g[:, None], _NEG)
    cmax = jnp.max(masked, axis=0)
    m_old = m_ref[0, :]
    m_new = jnp.maximum(m_old, cmax)
    alpha = jnp.exp(m_old - m_new)

    mrep = jax.lax.dot_general(ohf, m_new[:, None], (((1,), (0,)), ((), ())),
                               preferred_element_type=jnp.float32)[:, 0]
    e = jnp.exp(g - mrep)
    wgt = jnp.where(oh, e[:, None], 0.0)
    csum = jnp.sum(wgt, axis=0)

    m_ref[0, :] = m_new
    s_ref[0, :] = alpha * s_ref[0, :] + csum
    pacc = jax.lax.dot_general(wgt, h, (((0,), (0,)), ((), ())),
                               preferred_element_type=jnp.float32)
    acc_ref[...] = alpha[:, None] * acc_ref[...] + pacc

    @pl.when(i == k - 1)
    def _():
        s = s_ref[0, :]
        out_ref[...] = jnp.where(s[:, None] > 0,
                                 acc_ref[...] / s[:, None],
                                 0.0)


def kernel(h, segment_ids, attention_query, w):
    n = h.shape[0]
    k = n // _CHUNK
    wr = w.reshape(2, _HID).T
    seg3 = segment_ids.astype(jnp.int32).reshape(k, 1, _CHUNK)
    return pl.pallas_call(
        _body,
        grid=(k,),
        in_specs=[
            pl.BlockSpec((1, 1, _CHUNK), lambda i: (i, 0, 0)),
            pl.BlockSpec((_CHUNK, _HID), lambda i: (i, 0)),
            pl.BlockSpec((_HID, 2), lambda i: (0, 0)),
            pl.BlockSpec((_NSEG, _HID), lambda i: (0, 0)),
        ],
        out_specs=pl.BlockSpec((_NSEG, _HID), lambda i: (0, 0)),
        out_shape=jax.ShapeDtypeStruct((_NSEG, _HID), jnp.float32),
        scratch_shapes=[
            pltpu.VMEM((1, _NSEG), jnp.float32),
            pltpu.VMEM((1, _NSEG), jnp.float32),
            pltpu.VMEM((_NSEG, _HID), jnp.float32),
        ],
    )(seg3, h, wr, attention_query)
